# Initial kernel scaffold; baseline (speedup 1.0000x reference)
import jax
import jax.numpy as jnp
from jax.experimental import pallas as pl


def kernel(x, edge_index, batch, W1, b1, W2, b2, W3, b3, node_W, node_b, graph_W, graph_b):
    raise NotImplementedError("write your pallas kernel here")


# speedup vs baseline: 18.7287x; 18.7287x over previous
import functools

import jax
import jax.numpy as jnp
from jax import lax
from jax.experimental import pallas as pl
from jax.experimental.pallas import tpu as pltpu
from jax.experimental.pallas import tpu_sc as plsc

N = 10000
NP = 10240
E = 320000
D = 128
H = 96
HP = 128
G = 64

NC = 2
NS = 16
NW = NC * NS
EDGES_PER_TILE = E // NW
CHUNK = 125
NCHUNK = EDGES_PER_TILE // CHUNK
ROWS_PER_TILE = NP // NS

_f32 = jnp.float32


def _sc_mesh():
    return plsc.VectorSubcoreMesh(core_axis_name="c", subcore_axis_name="s")


@functools.partial(
    pl.kernel,
    out_type=(
        jax.ShapeDtypeStruct((NP,), _f32),
        jax.ShapeDtypeStruct((NP,), _f32),
    ),
    mesh=_sc_mesh(),
    scratch_types=[
        pltpu.VMEM((NCHUNK, CHUNK), jnp.int32),
        pltpu.VMEM((CHUNK,), _f32),
        pltpu.VMEM_SHARED((NP,), _f32),
    ],
)
def _sc_degree(d_hbm, zeros_hbm, ones_hbm, deg_a, deg_b, d_idx, ones_v, deg_sh):
    c = lax.axis_index("c")
    s = lax.axis_index("s")
    wid = s * NC + c

    @pl.when(s == 0)
    def _():
        pltpu.sync_copy(zeros_hbm, deg_sh)
    pltpu.sync_copy(ones_hbm, ones_v)
    pltpu.sync_copy(d_hbm.at[pl.ds(wid * NCHUNK, NCHUNK)], d_idx)
    plsc.subcore_barrier()

    def body(j, carry):
        pltpu.sync_copy(ones_v, deg_sh.at[d_idx.at[j]], add=True)
        return carry

    lax.fori_loop(0, NCHUNK, body, 0)
    plsc.subcore_barrier()

    @pl.when((s == 0) & (c == 0))
    def _():
        pltpu.sync_copy(deg_sh, deg_a)

    @pl.when((s == 0) & (c == 1))
    def _():
        pltpu.sync_copy(deg_sh, deg_b)


@functools.partial(
    pl.kernel,
    out_type=(
        jax.ShapeDtypeStruct((NP, HP), _f32),
        jax.ShapeDtypeStruct((NP, HP), _f32),
    ),
    mesh=_sc_mesh(),
    scratch_types=[
        pltpu.VMEM((NCHUNK, CHUNK), jnp.int32),
        pltpu.VMEM((NCHUNK, CHUNK), jnp.int32),
        pltpu.VMEM((CHUNK, HP), _f32),
        pltpu.VMEM_SHARED((NP, HP), _f32),
        pltpu.SemaphoreType.DMA,
    ],
)
def _sc_aggregate(p_hbm, zeros_hbm, s_hbm, d_hbm, out_a, out_b,
                  s_idx, d_idx, rows, acc_sh, sem):
    c = lax.axis_index("c")
    s = lax.axis_index("s")
    wid = s * NC + c
    row0 = s * ROWS_PER_TILE

    @pl.when(c == 0)
    def _():
        pltpu.sync_copy(p_hbm.at[pl.ds(row0, ROWS_PER_TILE)],
                        acc_sh.at[pl.ds(row0, ROWS_PER_TILE)])

    @pl.when(c == 1)
    def _():
        pltpu.sync_copy(zeros_hbm.at[pl.ds(row0, ROWS_PER_TILE)],
                        acc_sh.at[pl.ds(row0, ROWS_PER_TILE)])

    pltpu.sync_copy(s_hbm.at[pl.ds(wid * NCHUNK, NCHUNK)], s_idx)
    pltpu.sync_copy(d_hbm.at[pl.ds(wid * NCHUNK, NCHUNK)], d_idx)
    plsc.subcore_barrier()

    def body(j, carry):
        pltpu.async_copy(p_hbm.at[s_idx.at[j]], rows, sem).wait()
        pltpu.sync_copy(rows, acc_sh.at[d_idx.at[j]], add=True)
        return carry

    lax.fori_loop(0, NCHUNK, body, 0)
    plsc.subcore_barrier()

    @pl.when(c == 0)
    def _():
        pltpu.sync_copy(acc_sh.at[pl.ds(row0, ROWS_PER_TILE)],
                        out_a.at[pl.ds(row0, ROWS_PER_TILE)])

    @pl.when(c == 1)
    def _():
        pltpu.sync_copy(acc_sh.at[pl.ds(row0, ROWS_PER_TILE)],
                        out_b.at[pl.ds(row0, ROWS_PER_TILE)])


def _tc_first(x_ref, w_ref, dega_ref, degb_ref, p_ref, dinv_ref):
    deg = dega_ref[...] + degb_ref[...] + 1.0
    dinv = lax.rsqrt(deg)
    dinv_ref[...] = dinv
    p_ref[...] = dinv * jnp.dot(x_ref[...], w_ref[...],
                                preferred_element_type=_f32)


def _tc_mid(aa_ref, ab_ref, dinv_ref, b_ref, w_ref, p_ref):
    dinv = dinv_ref[...]
    h = jax.nn.relu(dinv * (aa_ref[...] + ab_ref[...]) + b_ref[...])
    p_ref[...] = dinv * jnp.dot(h, w_ref[...], preferred_element_type=_f32)


def _tc_h3(aa_ref, ab_ref, dinv_ref, b_ref, h_ref):
    dinv = dinv_ref[...]
    h_ref[...] = jax.nn.relu(
        dinv * (aa_ref[...] + ab_ref[...]) + b_ref[...])


def _tc_segmax(h_ref, batch_ref, hm_ref):
    bm = batch_ref[...]
    hm = h_ref[...]
    k = 1
    while k < NP:
        hs = jnp.concatenate([jnp.zeros((k, HP), _f32), hm[:-k]], axis=0)
        bs = jnp.concatenate(
            [jnp.full((k, 1), -1, jnp.int32), bm[:-k]], axis=0)
        hm = jnp.where(bs == bm, jnp.maximum(hm, hs), hm)
        k *= 2
    hm_ref[...] = hm


def _tc_final(h_ref, hm_ref, nw_ref, nb_ref,
              gw1_ref, gw2_ref, gb_ref, batch_ref,
              graph_ref, node_ref):
    h = h_ref[...][:, :H]

    node_ref[...] = jnp.dot(h, nw_ref[...], preferred_element_type=_f32) \
        + nb_ref[...]

    bm = batch_ref[...]
    gids = lax.broadcasted_iota(jnp.int32, (1, G), 1)
    onehot = (bm == gids).astype(_f32)
    contract = (((0,), (0,)), ((), ()))
    sums = lax.dot_general(onehot, h, contract,
                           preferred_element_type=_f32)
    ones_col = jnp.ones((NP, 1), _f32)
    counts = lax.dot_general(onehot, ones_col, contract,
                             preferred_element_type=_f32)
    mean_p = sums / jnp.maximum(counts, 1.0)

    b_next = jnp.concatenate(
        [bm[1:], jnp.full((1, 1), -2, jnp.int32)], axis=0)
    is_end = (b_next != bm).astype(_f32)
    oh_end = onehot * is_end
    max_p = lax.dot_general(oh_end, hm_ref[...][:, :H], contract,
                            preferred_element_type=_f32)

    graph_ref[...] = (
        jnp.dot(mean_p, gw1_ref[...], preferred_element_type=_f32)
        + jnp.dot(max_p, gw2_ref[...], preferred_element_type=_f32)
        + gb_ref[...]
    )


def _tc_call(body, out_shapes, *args):
    return pl.pallas_call(body, out_shape=out_shapes)(*args)


def _pad_cols(w):
    return jnp.concatenate(
        [w, jnp.zeros((w.shape[0], HP - w.shape[1]), _f32)], axis=1)


def _pad_rows(w):
    return jnp.concatenate(
        [w, jnp.zeros((HP - w.shape[0], w.shape[1]), _f32)], axis=0)


@jax.jit
def kernel(x, edge_index, batch, W1, b1, W2, b2, W3, b3,
           node_W, node_b, graph_W, graph_b):
    s_r = edge_index[0].reshape(NW * NCHUNK, CHUNK)
    d_r = edge_index[1].reshape(NW * NCHUNK, CHUNK)
    zeros_nh = jnp.zeros((NP, HP), _f32)
    zeros_n = jnp.zeros((NP,), _f32)
    ones_c = jnp.ones((CHUNK,), _f32)
    xp = jnp.concatenate([x, jnp.zeros((NP - N, D), _f32)], axis=0)
    batch2 = jnp.concatenate(
        [batch, jnp.full((NP - N,), -1, jnp.int32)]).reshape(NP, 1)
    W1p = _pad_cols(W1)
    W2p = _pad_rows(_pad_cols(W2))
    W3p = _pad_rows(_pad_cols(W3))
    b1p = _pad_cols(b1.reshape(1, H))
    b2p = _pad_cols(b2.reshape(1, H))
    b3p = _pad_cols(b3.reshape(1, H))
    gw1 = graph_W[:H]
    gw2 = graph_W[H:]

    deg_a, deg_b = _sc_degree(d_r, zeros_n, ones_c)

    p1, dinv = _tc_call(
        _tc_first,
        (jax.ShapeDtypeStruct((NP, HP), _f32),
         jax.ShapeDtypeStruct((NP, 1), _f32)),
        xp, W1p, deg_a.reshape(NP, 1), deg_b.reshape(NP, 1))

    a1, b1_ = _sc_aggregate(p1, zeros_nh, s_r, d_r)
    p2 = _tc_call(
        _tc_mid, jax.ShapeDtypeStruct((NP, HP), _f32),
        a1, b1_, dinv, b1p, W2p)

    a2, b2_ = _sc_aggregate(p2, zeros_nh, s_r, d_r)
    p3 = _tc_call(
        _tc_mid, jax.ShapeDtypeStruct((NP, HP), _f32),
        a2, b2_, dinv, b2p, W3p)

    a3, b3_ = _sc_aggregate(p3, zeros_nh, s_r, d_r)
    h3 = _tc_call(
        _tc_h3, jax.ShapeDtypeStruct((NP, HP), _f32),
        a3, b3_, dinv, b3p)

    hm3 = _tc_call(
        _tc_segmax, jax.ShapeDtypeStruct((NP, HP), _f32),
        h3, batch2)

    graph_logits, node_logits = _tc_call(
        _tc_final,
        (jax.ShapeDtypeStruct((G, 1), _f32),
         jax.ShapeDtypeStruct((NP, 1), _f32)),
        h3, hm3, node_W, node_b.reshape(1, 1),
        gw1, gw2, graph_b.reshape(1, 1), batch2)

    return graph_logits.reshape(-1), node_logits[:N].reshape(-1)


# speedup vs baseline: 23.2824x; 1.2431x over previous
import functools

import jax
import jax.numpy as jnp
from jax import lax
from jax.experimental import pallas as pl
from jax.experimental.pallas import tpu as pltpu
from jax.experimental.pallas import tpu_sc as plsc

N = 10000
NP = 10240
E = 320000
D = 128
H = 96
HP = 128
G = 64

NC = 2
NS = 16
NW = NC * NS
EDGES_PER_TILE = E // NW
CHUNK = 125
NCHUNK = EDGES_PER_TILE // CHUNK
ROWS_PER_TILE = NP // NS

_f32 = jnp.float32


def _sc_mesh():
    return plsc.VectorSubcoreMesh(core_axis_name="c", subcore_axis_name="s")


@functools.partial(
    pl.kernel,
    out_type=(
        jax.ShapeDtypeStruct((NP,), _f32),
        jax.ShapeDtypeStruct((NP,), _f32),
    ),
    mesh=_sc_mesh(),
    scratch_types=[
        pltpu.VMEM((NCHUNK, CHUNK), jnp.int32),
        pltpu.VMEM((CHUNK,), _f32),
        pltpu.VMEM_SHARED((NP,), _f32),
    ],
)
def _sc_degree(d_hbm, zeros_hbm, ones_hbm, deg_a, deg_b, d_idx, ones_v, deg_sh):
    c = lax.axis_index("c")
    s = lax.axis_index("s")
    wid = s * NC + c

    @pl.when(s == 0)
    def _():
        pltpu.sync_copy(zeros_hbm, deg_sh)
    pltpu.sync_copy(ones_hbm, ones_v)
    pltpu.sync_copy(d_hbm.at[pl.ds(wid * NCHUNK, NCHUNK)], d_idx)
    plsc.subcore_barrier()

    def body(j, carry):
        pltpu.sync_copy(ones_v, deg_sh.at[d_idx.at[j]], add=True)
        return carry

    lax.fori_loop(0, NCHUNK, body, 0)
    plsc.subcore_barrier()

    @pl.when((s == 0) & (c == 0))
    def _():
        pltpu.sync_copy(deg_sh, deg_a)

    @pl.when((s == 0) & (c == 1))
    def _():
        pltpu.sync_copy(deg_sh, deg_b)


@functools.partial(
    pl.kernel,
    out_type=(
        jax.ShapeDtypeStruct((NP, HP), _f32),
        jax.ShapeDtypeStruct((NP, HP), _f32),
    ),
    mesh=_sc_mesh(),
    scratch_types=[
        pltpu.VMEM((NCHUNK // 2, CHUNK), jnp.int32),
        pltpu.VMEM((NCHUNK // 2, CHUNK), jnp.int32),
        pltpu.VMEM((CHUNK, HP), _f32),
        pltpu.VMEM((CHUNK, HP), _f32),
        pltpu.VMEM_SHARED((NP, HP), _f32),
        pltpu.SemaphoreType.DMA,
        pltpu.SemaphoreType.DMA,
    ],
)
def _sc_aggregate(p_hbm, zeros_hbm, s_hbm, d_hbm, out_a, out_b,
                  s_idx, d_idx, rows0, rows1, acc_sh, sem0, sem1):
    c = lax.axis_index("c")
    s = lax.axis_index("s")
    wid = s * NC + c
    row0 = s * ROWS_PER_TILE

    @pl.when(c == 0)
    def _():
        pltpu.sync_copy(p_hbm.at[pl.ds(row0, ROWS_PER_TILE)],
                        acc_sh.at[pl.ds(row0, ROWS_PER_TILE)])

    @pl.when(c == 1)
    def _():
        pltpu.sync_copy(zeros_hbm.at[pl.ds(row0, ROWS_PER_TILE)],
                        acc_sh.at[pl.ds(row0, ROWS_PER_TILE)])

    plsc.subcore_barrier()

    HC = NCHUNK // 2
    for half in range(2):
        base = wid * NCHUNK + half * HC
        pltpu.sync_copy(s_hbm.at[pl.ds(base, HC)], s_idx)
        pltpu.sync_copy(d_hbm.at[pl.ds(base, HC)], d_idx)
        pltpu.async_copy(p_hbm.at[s_idx.at[0]], rows0, sem0)

        def body(i, carry):
            g = 2 * i
            pltpu.make_async_copy(p_hbm.at[s_idx.at[g]], rows0, sem0).wait()
            pltpu.async_copy(p_hbm.at[s_idx.at[g + 1]], rows1, sem1)
            pltpu.sync_copy(rows0, acc_sh.at[d_idx.at[g]], add=True)
            pltpu.make_async_copy(
                p_hbm.at[s_idx.at[g + 1]], rows1, sem1).wait()

            @pl.when(g + 2 < HC)
            def _():
                pltpu.async_copy(p_hbm.at[s_idx.at[g + 2]], rows0, sem0)
            pltpu.sync_copy(rows1, acc_sh.at[d_idx.at[g + 1]], add=True)
            return carry

        lax.fori_loop(0, HC // 2, body, 0)
    plsc.subcore_barrier()

    @pl.when(c == 0)
    def _():
        pltpu.sync_copy(acc_sh.at[pl.ds(row0, ROWS_PER_TILE)],
                        out_a.at[pl.ds(row0, ROWS_PER_TILE)])

    @pl.when(c == 1)
    def _():
        pltpu.sync_copy(acc_sh.at[pl.ds(row0, ROWS_PER_TILE)],
                        out_b.at[pl.ds(row0, ROWS_PER_TILE)])


def _tc_first(x_ref, w_ref, dega_ref, degb_ref, p_ref, dinv_ref):
    deg = dega_ref[...] + degb_ref[...] + 1.0
    dinv = lax.rsqrt(deg)
    dinv_ref[...] = dinv
    p_ref[...] = dinv * jnp.dot(x_ref[...], w_ref[...],
                                preferred_element_type=_f32)


def _tc_mid(aa_ref, ab_ref, dinv_ref, b_ref, w_ref, p_ref):
    dinv = dinv_ref[...]
    h = jax.nn.relu(dinv * (aa_ref[...] + ab_ref[...]) + b_ref[...])
    p_ref[...] = dinv * jnp.dot(h, w_ref[...], preferred_element_type=_f32)


def _tc_h3(aa_ref, ab_ref, dinv_ref, b_ref, h_ref):
    dinv = dinv_ref[...]
    h_ref[...] = jax.nn.relu(
        dinv * (aa_ref[...] + ab_ref[...]) + b_ref[...])


def _tc_segmax(h_ref, batch_ref, hm_ref):
    bm = batch_ref[...]
    hm = h_ref[...]
    k = 1
    while k < NP:
        hs = jnp.concatenate([jnp.zeros((k, HP), _f32), hm[:-k]], axis=0)
        bs = jnp.concatenate(
            [jnp.full((k, 1), -1, jnp.int32), bm[:-k]], axis=0)
        hm = jnp.where(bs == bm, jnp.maximum(hm, hs), hm)
        k *= 2
    hm_ref[...] = hm


def _tc_final(h_ref, hm_ref, nw_ref, nb_ref,
              gw1_ref, gw2_ref, gb_ref, batch_ref,
              graph_ref, node_ref):
    h = h_ref[...][:, :H]

    node_ref[...] = jnp.dot(h, nw_ref[...], preferred_element_type=_f32) \
        + nb_ref[...]

    bm = batch_ref[...]
    gids = lax.broadcasted_iota(jnp.int32, (1, G), 1)
    onehot = (bm == gids).astype(_f32)
    contract = (((0,), (0,)), ((), ()))
    sums = lax.dot_general(onehot, h, contract,
                           preferred_element_type=_f32)
    ones_col = jnp.ones((NP, 1), _f32)
    counts = lax.dot_general(onehot, ones_col, contract,
                             preferred_element_type=_f32)
    mean_p = sums / jnp.maximum(counts, 1.0)

    b_next = jnp.concatenate(
        [bm[1:], jnp.full((1, 1), -2, jnp.int32)], axis=0)
    is_end = (b_next != bm).astype(_f32)
    oh_end = onehot * is_end
    max_p = lax.dot_general(oh_end, hm_ref[...][:, :H], contract,
                            preferred_element_type=_f32)

    graph_ref[...] = (
        jnp.dot(mean_p, gw1_ref[...], preferred_element_type=_f32)
        + jnp.dot(max_p, gw2_ref[...], preferred_element_type=_f32)
        + gb_ref[...]
    )


def _tc_call(body, out_shapes, *args):
    return pl.pallas_call(body, out_shape=out_shapes)(*args)


def _pad_cols(w):
    return jnp.concatenate(
        [w, jnp.zeros((w.shape[0], HP - w.shape[1]), _f32)], axis=1)


def _pad_rows(w):
    return jnp.concatenate(
        [w, jnp.zeros((HP - w.shape[0], w.shape[1]), _f32)], axis=0)


@jax.jit
def kernel(x, edge_index, batch, W1, b1, W2, b2, W3, b3,
           node_W, node_b, graph_W, graph_b):
    s_r = edge_index[0].reshape(NW * NCHUNK, CHUNK)
    d_r = edge_index[1].reshape(NW * NCHUNK, CHUNK)
    zeros_nh = jnp.zeros((NP, HP), _f32)
    zeros_n = jnp.zeros((NP,), _f32)
    ones_c = jnp.ones((CHUNK,), _f32)
    xp = jnp.concatenate([x, jnp.zeros((NP - N, D), _f32)], axis=0)
    batch2 = jnp.concatenate(
        [batch, jnp.full((NP - N,), -1, jnp.int32)]).reshape(NP, 1)
    W1p = _pad_cols(W1)
    W2p = _pad_rows(_pad_cols(W2))
    W3p = _pad_rows(_pad_cols(W3))
    b1p = _pad_cols(b1.reshape(1, H))
    b2p = _pad_cols(b2.reshape(1, H))
    b3p = _pad_cols(b3.reshape(1, H))
    gw1 = graph_W[:H]
    gw2 = graph_W[H:]

    deg_a, deg_b = _sc_degree(d_r, zeros_n, ones_c)

    p1, dinv = _tc_call(
        _tc_first,
        (jax.ShapeDtypeStruct((NP, HP), _f32),
         jax.ShapeDtypeStruct((NP, 1), _f32)),
        xp, W1p, deg_a.reshape(NP, 1), deg_b.reshape(NP, 1))

    a1, b1_ = _sc_aggregate(p1, zeros_nh, s_r, d_r)
    p2 = _tc_call(
        _tc_mid, jax.ShapeDtypeStruct((NP, HP), _f32),
        a1, b1_, dinv, b1p, W2p)

    a2, b2_ = _sc_aggregate(p2, zeros_nh, s_r, d_r)
    p3 = _tc_call(
        _tc_mid, jax.ShapeDtypeStruct((NP, HP), _f32),
        a2, b2_, dinv, b2p, W3p)

    a3, b3_ = _sc_aggregate(p3, zeros_nh, s_r, d_r)
    h3 = _tc_call(
        _tc_h3, jax.ShapeDtypeStruct((NP, HP), _f32),
        a3, b3_, dinv, b3p)

    hm3 = _tc_call(
        _tc_segmax, jax.ShapeDtypeStruct((NP, HP), _f32),
        h3, batch2)

    graph_logits, node_logits = _tc_call(
        _tc_final,
        (jax.ShapeDtypeStruct((G, 1), _f32),
         jax.ShapeDtypeStruct((NP, 1), _f32)),
        h3, hm3, node_W, node_b.reshape(1, 1),
        gw1, gw2, graph_b.reshape(1, 1), batch2)

    return graph_logits.reshape(-1), node_logits[:N].reshape(-1)


# speedup vs baseline: 23.5335x; 1.0108x over previous
import functools

import jax
import jax.numpy as jnp
from jax import lax
from jax.experimental import pallas as pl
from jax.experimental.pallas import tpu as pltpu
from jax.experimental.pallas import tpu_sc as plsc

N = 10000
NP = 10240
E = 320000
EP = 327680
D = 128
H = 96
HP = 128
G = 64

NC = 2
NS = 16
NW = NC * NS
EDGES_PER_TILE = EP // NW
CHUNK = 128
NCHUNK = EDGES_PER_TILE // CHUNK
NBUF = 2
ROWS_PER_TILE = NP // NS

_f32 = jnp.float32


def _sc_mesh():
    return plsc.VectorSubcoreMesh(core_axis_name="c", subcore_axis_name="s")


@functools.partial(
    pl.kernel,
    out_type=(
        jax.ShapeDtypeStruct((NP,), _f32),
        jax.ShapeDtypeStruct((NP,), _f32),
    ),
    mesh=_sc_mesh(),
    scratch_types=[
        pltpu.VMEM((NCHUNK, CHUNK), jnp.int32),
        pltpu.VMEM((CHUNK,), _f32),
        pltpu.VMEM_SHARED((NP,), _f32),
    ],
)
def _sc_degree(d_hbm, zeros_hbm, ones_hbm, deg_a, deg_b, d_idx, ones_v, deg_sh):
    c = lax.axis_index("c")
    s = lax.axis_index("s")
    wid = s * NC + c

    @pl.when(s == 0)
    def _():
        pltpu.sync_copy(zeros_hbm, deg_sh)
    pltpu.sync_copy(ones_hbm, ones_v)
    pltpu.sync_copy(d_hbm.at[pl.ds(wid * NCHUNK, NCHUNK)], d_idx)
    plsc.subcore_barrier()

    def body(j, carry):
        pltpu.sync_copy(ones_v, deg_sh.at[d_idx.at[j]], add=True)
        return carry

    lax.fori_loop(0, NCHUNK, body, 0)
    plsc.subcore_barrier()

    @pl.when((s == 0) & (c == 0))
    def _():
        pltpu.sync_copy(deg_sh, deg_a)

    @pl.when((s == 0) & (c == 1))
    def _():
        pltpu.sync_copy(deg_sh, deg_b)


@functools.partial(
    pl.kernel,
    out_type=(
        jax.ShapeDtypeStruct((NP, HP), _f32),
        jax.ShapeDtypeStruct((NP, HP), _f32),
    ),
    mesh=_sc_mesh(),
    scratch_types=[
        pltpu.VMEM((NCHUNK, CHUNK), jnp.int32),
        pltpu.VMEM((NBUF, CHUNK), jnp.int32),
        pltpu.VMEM((NBUF, CHUNK), jnp.int32),
        pltpu.VMEM((NBUF, CHUNK, HP), _f32),
        pltpu.VMEM_SHARED((NP, HP), _f32),
    ] + [pltpu.SemaphoreType.DMA] * (2 * NBUF),
)
def _sc_aggregate(p_hbm, zeros_hbm, sd_hbm, out_a, out_b,
                  sd_idx, s_buf, d_buf, rows, acc_sh, *sems):
    c = lax.axis_index("c")
    s = lax.axis_index("s")
    wid = s * NC + c
    row0 = s * ROWS_PER_TILE
    gsem = sems[:NBUF]
    ssem = sems[NBUF:]

    @pl.when(c == 0)
    def _():
        pltpu.sync_copy(p_hbm.at[pl.ds(row0, ROWS_PER_TILE)],
                        acc_sh.at[pl.ds(row0, ROWS_PER_TILE)])

    @pl.when(c == 1)
    def _():
        pltpu.sync_copy(zeros_hbm.at[pl.ds(row0, ROWS_PER_TILE)],
                        acc_sh.at[pl.ds(row0, ROWS_PER_TILE)])

    pltpu.sync_copy(sd_hbm.at[pl.ds(wid * NCHUNK, NCHUNK)], sd_idx)
    plsc.subcore_barrier()

    def unpack(j, b):
        for q in range(CHUNK // 16):
            v = sd_idx[j, pl.ds(q * 16, 16)]
            s_buf[b, pl.ds(q * 16, 16)] = lax.bitwise_and(v, 0x3FFF)
            d_buf[b, pl.ds(q * 16, 16)] = lax.shift_right_logical(v, 14)

    unpack(0, 0)
    pltpu.async_copy(p_hbm.at[s_buf.at[0]], rows.at[0], gsem[0])

    def group(i0, carry):
        for b in range(NBUF):
            j = NBUF * i0 + b
            pltpu.make_async_copy(
                p_hbm.at[s_buf.at[b]], rows.at[b], gsem[b]).wait()
            pltpu.async_copy(rows.at[b], acc_sh.at[d_buf.at[b]],
                             ssem[b], add=True)
            bn = 1 - b

            def prep(j=j, bn=bn):
                @pl.when(j >= 1)
                def _():
                    pltpu.make_async_copy(
                        rows.at[bn], acc_sh.at[d_buf.at[bn]],
                        ssem[bn]).wait()
                unpack(j + 1, bn)
                pltpu.async_copy(
                    p_hbm.at[s_buf.at[bn]], rows.at[bn], gsem[bn])

            if b == 0:
                prep()
            else:
                pl.when(j + 1 < NCHUNK)(prep)
        return carry

    lax.fori_loop(0, NCHUNK // NBUF, group, 0)
    for b in range(NBUF):
        pltpu.make_async_copy(
            rows.at[b], acc_sh.at[d_buf.at[b]], ssem[b]).wait()
    plsc.subcore_barrier()

    @pl.when(c == 0)
    def _():
        pltpu.sync_copy(acc_sh.at[pl.ds(row0, ROWS_PER_TILE)],
                        out_a.at[pl.ds(row0, ROWS_PER_TILE)])

    @pl.when(c == 1)
    def _():
        pltpu.sync_copy(acc_sh.at[pl.ds(row0, ROWS_PER_TILE)],
                        out_b.at[pl.ds(row0, ROWS_PER_TILE)])


def _tc_first(x_ref, w_ref, dega_ref, degb_ref, p_ref, dinv_ref):
    deg = dega_ref[...] + degb_ref[...] + 1.0
    dinv = lax.rsqrt(deg)
    dinv_ref[...] = dinv
    p_ref[...] = dinv * jnp.dot(x_ref[...], w_ref[...],
                                preferred_element_type=_f32)


def _tc_mid(aa_ref, ab_ref, dinv_ref, b_ref, w_ref, p_ref):
    dinv = dinv_ref[...]
    h = jax.nn.relu(dinv * (aa_ref[...] + ab_ref[...]) + b_ref[...])
    p_ref[...] = dinv * jnp.dot(h, w_ref[...], preferred_element_type=_f32)


def _tc_h3(aa_ref, ab_ref, dinv_ref, b_ref, h_ref):
    dinv = dinv_ref[...]
    h_ref[...] = jax.nn.relu(
        dinv * (aa_ref[...] + ab_ref[...]) + b_ref[...])


def _tc_segmax(h_ref, batch_ref, hm_ref):
    bm = batch_ref[...]
    hm = h_ref[...]
    k = 1
    while k < NP:
        hs = jnp.concatenate([jnp.zeros((k, HP), _f32), hm[:-k]], axis=0)
        bs = jnp.concatenate(
            [jnp.full((k, 1), -1, jnp.int32), bm[:-k]], axis=0)
        hm = jnp.where(bs == bm, jnp.maximum(hm, hs), hm)
        k *= 2
    hm_ref[...] = hm


def _tc_final(h_ref, hm_ref, nw_ref, nb_ref,
              gw1_ref, gw2_ref, gb_ref, batch_ref,
              graph_ref, node_ref):
    h = h_ref[...][:, :H]

    node_ref[...] = jnp.dot(h, nw_ref[...], preferred_element_type=_f32) \
        + nb_ref[...]

    bm = batch_ref[...]
    gids = lax.broadcasted_iota(jnp.int32, (1, G), 1)
    onehot = (bm == gids).astype(_f32)
    contract = (((0,), (0,)), ((), ()))
    sums = lax.dot_general(onehot, h, contract,
                           preferred_element_type=_f32)
    ones_col = jnp.ones((NP, 1), _f32)
    counts = lax.dot_general(onehot, ones_col, contract,
                             preferred_element_type=_f32)
    mean_p = sums / jnp.maximum(counts, 1.0)

    b_next = jnp.concatenate(
        [bm[1:], jnp.full((1, 1), -2, jnp.int32)], axis=0)
    is_end = (b_next != bm).astype(_f32)
    oh_end = onehot * is_end
    max_p = lax.dot_general(oh_end, hm_ref[...][:, :H], contract,
                            preferred_element_type=_f32)

    graph_ref[...] = (
        jnp.dot(mean_p, gw1_ref[...], preferred_element_type=_f32)
        + jnp.dot(max_p, gw2_ref[...], preferred_element_type=_f32)
        + gb_ref[...]
    )


def _tc_call(body, out_shapes, *args):
    return pl.pallas_call(body, out_shape=out_shapes)(*args)


def _pad_cols(w):
    return jnp.concatenate(
        [w, jnp.zeros((w.shape[0], HP - w.shape[1]), _f32)], axis=1)


def _pad_rows(w):
    return jnp.concatenate(
        [w, jnp.zeros((HP - w.shape[0], w.shape[1]), _f32)], axis=0)


@jax.jit
def kernel(x, edge_index, batch, W1, b1, W2, b2, W3, b3,
           node_W, node_b, graph_W, graph_b):
    npad = EP - E
    pad_s = (jnp.arange(npad, dtype=jnp.int32) * 131) % N
    pad_d = N + (jnp.arange(npad, dtype=jnp.int32) % (NP - N))
    s_full = jnp.concatenate([edge_index[0], pad_s])
    d_full = jnp.concatenate([edge_index[1], pad_d])
    sd_r = (s_full | (d_full << 14)).reshape(NW * NCHUNK, CHUNK)
    d_r = d_full.reshape(NW * NCHUNK, CHUNK)
    zeros_nh = jnp.zeros((NP, HP), _f32)
    zeros_n = jnp.zeros((NP,), _f32)
    ones_c = jnp.ones((CHUNK,), _f32)
    xp = jnp.concatenate([x, jnp.zeros((NP - N, D), _f32)], axis=0)
    batch2 = jnp.concatenate(
        [batch, jnp.full((NP - N,), -1, jnp.int32)]).reshape(NP, 1)
    W1p = _pad_cols(W1)
    W2p = _pad_rows(_pad_cols(W2))
    W3p = _pad_rows(_pad_cols(W3))
    b1p = _pad_cols(b1.reshape(1, H))
    b2p = _pad_cols(b2.reshape(1, H))
    b3p = _pad_cols(b3.reshape(1, H))
    gw1 = graph_W[:H]
    gw2 = graph_W[H:]

    deg_a, deg_b = _sc_degree(d_r, zeros_n, ones_c)

    p1, dinv = _tc_call(
        _tc_first,
        (jax.ShapeDtypeStruct((NP, HP), _f32),
         jax.ShapeDtypeStruct((NP, 1), _f32)),
        xp, W1p, deg_a.reshape(NP, 1), deg_b.reshape(NP, 1))

    a1, b1_ = _sc_aggregate(p1, zeros_nh, sd_r)
    p2 = _tc_call(
        _tc_mid, jax.ShapeDtypeStruct((NP, HP), _f32),
        a1, b1_, dinv, b1p, W2p)

    a2, b2_ = _sc_aggregate(p2, zeros_nh, sd_r)
    p3 = _tc_call(
        _tc_mid, jax.ShapeDtypeStruct((NP, HP), _f32),
        a2, b2_, dinv, b2p, W3p)

    a3, b3_ = _sc_aggregate(p3, zeros_nh, sd_r)
    h3 = _tc_call(
        _tc_h3, jax.ShapeDtypeStruct((NP, HP), _f32),
        a3, b3_, dinv, b3p)

    hm3 = _tc_call(
        _tc_segmax, jax.ShapeDtypeStruct((NP, HP), _f32),
        h3, batch2)

    graph_logits, node_logits = _tc_call(
        _tc_final,
        (jax.ShapeDtypeStruct((G, 1), _f32),
         jax.ShapeDtypeStruct((NP, 1), _f32)),
        h3, hm3, node_W, node_b.reshape(1, 1),
        gw1, gw2, graph_b.reshape(1, 1), batch2)

    return graph_logits.reshape(-1), node_logits[:N].reshape(-1)


# speedup vs baseline: 23.6844x; 1.0064x over previous
import functools

import jax
import jax.numpy as jnp
from jax import lax
from jax.experimental import pallas as pl
from jax.experimental.pallas import tpu as pltpu
from jax.experimental.pallas import tpu_sc as plsc

N = 10000
NP = 10240
E = 320000
EP = 327680
D = 128
H = 96
HP = 128
G = 64

NC = 2
NS = 16
NW = NC * NS
EDGES_PER_TILE = EP // NW
CHUNK = 128
NCHUNK = EDGES_PER_TILE // CHUNK
NBUF = 2
ROWS_PER_TILE = NP // NS

_f32 = jnp.float32


def _sc_mesh():
    return plsc.VectorSubcoreMesh(core_axis_name="c", subcore_axis_name="s")


@functools.partial(
    pl.kernel,
    out_type=(
        jax.ShapeDtypeStruct((NP,), _f32),
        jax.ShapeDtypeStruct((NP,), _f32),
    ),
    mesh=_sc_mesh(),
    scratch_types=[
        pltpu.VMEM((NCHUNK, CHUNK), jnp.int32),
        pltpu.VMEM((CHUNK,), _f32),
        pltpu.VMEM_SHARED((NP,), _f32),
    ],
)
def _sc_degree(d_hbm, zeros_hbm, ones_hbm, deg_a, deg_b, d_idx, ones_v, deg_sh):
    c = lax.axis_index("c")
    s = lax.axis_index("s")
    wid = s * NC + c

    @pl.when(s == 0)
    def _():
        pltpu.sync_copy(zeros_hbm, deg_sh)
    pltpu.sync_copy(ones_hbm, ones_v)
    pltpu.sync_copy(d_hbm.at[pl.ds(wid * NCHUNK, NCHUNK)], d_idx)
    plsc.subcore_barrier()

    def body(j, carry):
        pltpu.sync_copy(ones_v, deg_sh.at[d_idx.at[j]], add=True)
        return carry

    lax.fori_loop(0, NCHUNK, body, 0)
    plsc.subcore_barrier()

    @pl.when((s == 0) & (c == 0))
    def _():
        pltpu.sync_copy(deg_sh, deg_a)

    @pl.when((s == 0) & (c == 1))
    def _():
        pltpu.sync_copy(deg_sh, deg_b)


@functools.partial(
    pl.kernel,
    out_type=(
        jax.ShapeDtypeStruct((NP, HP), _f32),
        jax.ShapeDtypeStruct((NP, HP), _f32),
    ),
    mesh=_sc_mesh(),
    scratch_types=[
        pltpu.VMEM((NCHUNK, CHUNK), jnp.int32),
        pltpu.VMEM((NBUF, CHUNK), jnp.int32),
        pltpu.VMEM((NBUF, CHUNK), jnp.int32),
        pltpu.VMEM((NBUF, CHUNK, HP), _f32),
        pltpu.VMEM_SHARED((NP, HP), _f32),
    ] + [pltpu.SemaphoreType.DMA] * (2 * NBUF),
)
def _sc_aggregate(p_hbm, zeros_hbm, sd_hbm, out_a, out_b,
                  sd_idx, s_buf, d_buf, rows, acc_sh, *sems):
    c = lax.axis_index("c")
    s = lax.axis_index("s")
    wid = s * NC + c
    row0 = s * ROWS_PER_TILE
    gsem = sems[:NBUF]
    ssem = sems[NBUF:]

    @pl.when(c == 0)
    def _():
        pltpu.sync_copy(p_hbm.at[pl.ds(row0, ROWS_PER_TILE)],
                        acc_sh.at[pl.ds(row0, ROWS_PER_TILE)])

    @pl.when(c == 1)
    def _():
        pltpu.sync_copy(zeros_hbm.at[pl.ds(row0, ROWS_PER_TILE)],
                        acc_sh.at[pl.ds(row0, ROWS_PER_TILE)])

    pltpu.sync_copy(sd_hbm.at[pl.ds(wid * NCHUNK, NCHUNK)], sd_idx)
    plsc.subcore_barrier()

    def unpack(j, b):
        for q in range(CHUNK // 16):
            v = sd_idx[j, pl.ds(q * 16, 16)]
            s_buf[b, pl.ds(q * 16, 16)] = lax.bitwise_and(v, 0x3FFF)
            d_buf[b, pl.ds(q * 16, 16)] = lax.shift_right_logical(v, 14)

    unpack(0, 0)
    pltpu.async_copy(p_hbm.at[s_buf.at[0]], rows.at[0], gsem[0])

    def group(i0, carry):
        for b in range(NBUF):
            j = NBUF * i0 + b
            pltpu.make_async_copy(
                p_hbm.at[s_buf.at[b]], rows.at[b], gsem[b]).wait()
            pltpu.async_copy(rows.at[b], acc_sh.at[d_buf.at[b]],
                             ssem[b], add=True)
            bn = 1 - b

            def prep(j=j, bn=bn):
                @pl.when(j >= 1)
                def _():
                    pltpu.make_async_copy(
                        rows.at[bn], acc_sh.at[d_buf.at[bn]],
                        ssem[bn]).wait()
                unpack(j + 1, bn)
                pltpu.async_copy(
                    p_hbm.at[s_buf.at[bn]], rows.at[bn], gsem[bn])

            if b == 0:
                prep()
            else:
                pl.when(j + 1 < NCHUNK)(prep)
        return carry

    lax.fori_loop(0, NCHUNK // NBUF, group, 0)
    for b in range(NBUF):
        pltpu.make_async_copy(
            rows.at[b], acc_sh.at[d_buf.at[b]], ssem[b]).wait()
    plsc.subcore_barrier()

    @pl.when(c == 0)
    def _():
        pltpu.sync_copy(acc_sh.at[pl.ds(row0, ROWS_PER_TILE)],
                        out_a.at[pl.ds(row0, ROWS_PER_TILE)])

    @pl.when(c == 1)
    def _():
        pltpu.sync_copy(acc_sh.at[pl.ds(row0, ROWS_PER_TILE)],
                        out_b.at[pl.ds(row0, ROWS_PER_TILE)])


def _tc_first(x_ref, w_ref, dega_ref, degb_ref, p_ref, dinv_ref):
    deg = dega_ref[...] + degb_ref[...] + 1.0
    dinv = lax.rsqrt(deg)
    dinv_ref[...] = dinv
    p_ref[...] = dinv * jnp.dot(x_ref[...], w_ref[...],
                                preferred_element_type=_f32)


def _tc_mid(aa_ref, ab_ref, dinv_ref, b_ref, w_ref, p_ref):
    dinv = dinv_ref[...]
    h = jax.nn.relu(dinv * (aa_ref[...] + ab_ref[...]) + b_ref[...])
    p_ref[...] = dinv * jnp.dot(h, w_ref[...], preferred_element_type=_f32)


def _tc_h3(aa_ref, ab_ref, dinv_ref, b_ref, h_ref):
    dinv = dinv_ref[...]
    h_ref[...] = jax.nn.relu(
        dinv * (aa_ref[...] + ab_ref[...]) + b_ref[...])


def _tc_segmax(h_ref, batch_ref, hm_ref):
    bm = batch_ref[...]
    hm = h_ref[...]
    k = 1
    while k < NP:
        hs = jnp.concatenate([jnp.zeros((k, HP), _f32), hm[:-k]], axis=0)
        bs = jnp.concatenate(
            [jnp.full((k, 1), -1, jnp.int32), bm[:-k]], axis=0)
        hm = jnp.where(bs == bm, jnp.maximum(hm, hs), hm)
        k *= 2
    hm_ref[...] = hm


def _tc_final(h_ref, hm_ref, nw_ref, nb_ref,
              gw1_ref, gw2_ref, gb_ref, batch_ref,
              graph_ref, node_ref):
    h = h_ref[...][:, :H]

    node_ref[...] = lax.dot_general(
        nw_ref[...], h, (((0,), (1,)), ((), ())),
        preferred_element_type=_f32) + nb_ref[...]

    bm = batch_ref[...]
    gids = lax.broadcasted_iota(jnp.int32, (1, G), 1)
    onehot = (bm == gids).astype(_f32)
    contract = (((0,), (0,)), ((), ()))
    sums = lax.dot_general(onehot, h, contract,
                           preferred_element_type=_f32)
    ones_col = jnp.ones((NP, 1), _f32)
    counts = lax.dot_general(onehot, ones_col, contract,
                             preferred_element_type=_f32)
    mean_p = sums / jnp.maximum(counts, 1.0)

    b_next = jnp.concatenate(
        [bm[1:], jnp.full((1, 1), -2, jnp.int32)], axis=0)
    is_end = (b_next != bm).astype(_f32)
    oh_end = onehot * is_end
    max_p = lax.dot_general(oh_end, hm_ref[...][:, :H], contract,
                            preferred_element_type=_f32)

    graph_ref[...] = (
        jnp.dot(mean_p, gw1_ref[...], preferred_element_type=_f32)
        + jnp.dot(max_p, gw2_ref[...], preferred_element_type=_f32)
        + gb_ref[...]
    )


def _tc_call(body, out_shapes, *args):
    return pl.pallas_call(body, out_shape=out_shapes)(*args)


def _pad_cols(w):
    return jnp.concatenate(
        [w, jnp.zeros((w.shape[0], HP - w.shape[1]), _f32)], axis=1)


def _pad_rows(w):
    return jnp.concatenate(
        [w, jnp.zeros((HP - w.shape[0], w.shape[1]), _f32)], axis=0)


@jax.jit
def kernel(x, edge_index, batch, W1, b1, W2, b2, W3, b3,
           node_W, node_b, graph_W, graph_b):
    npad = EP - E
    pad_s = (jnp.arange(npad, dtype=jnp.int32) * 131) % N
    pad_d = N + (jnp.arange(npad, dtype=jnp.int32) % (NP - N))
    s_full = jnp.concatenate([edge_index[0], pad_s])
    d_full = jnp.concatenate([edge_index[1], pad_d])
    sd_r = (s_full | (d_full << 14)).reshape(NW * NCHUNK, CHUNK)
    d_r = d_full.reshape(NW * NCHUNK, CHUNK)
    zeros_nh = jnp.zeros((NP, HP), _f32)
    zeros_n = jnp.zeros((NP,), _f32)
    ones_c = jnp.ones((CHUNK,), _f32)
    xp = jnp.concatenate([x, jnp.zeros((NP - N, D), _f32)], axis=0)
    batch2 = jnp.concatenate(
        [batch, jnp.full((NP - N,), -1, jnp.int32)]).reshape(NP, 1)
    W1p = _pad_cols(W1)
    W2p = _pad_rows(_pad_cols(W2))
    W3p = _pad_rows(_pad_cols(W3))
    b1p = _pad_cols(b1.reshape(1, H))
    b2p = _pad_cols(b2.reshape(1, H))
    b3p = _pad_cols(b3.reshape(1, H))
    gw1 = graph_W[:H]
    gw2 = graph_W[H:]

    deg_a, deg_b = _sc_degree(d_r, zeros_n, ones_c)

    p1, dinv = _tc_call(
        _tc_first,
        (jax.ShapeDtypeStruct((NP, HP), _f32),
         jax.ShapeDtypeStruct((NP, 1), _f32)),
        xp, W1p, deg_a.reshape(NP, 1), deg_b.reshape(NP, 1))

    a1, b1_ = _sc_aggregate(p1, zeros_nh, sd_r)
    p2 = _tc_call(
        _tc_mid, jax.ShapeDtypeStruct((NP, HP), _f32),
        a1, b1_, dinv, b1p, W2p)

    a2, b2_ = _sc_aggregate(p2, zeros_nh, sd_r)
    p3 = _tc_call(
        _tc_mid, jax.ShapeDtypeStruct((NP, HP), _f32),
        a2, b2_, dinv, b2p, W3p)

    a3, b3_ = _sc_aggregate(p3, zeros_nh, sd_r)
    h3 = _tc_call(
        _tc_h3, jax.ShapeDtypeStruct((NP, HP), _f32),
        a3, b3_, dinv, b3p)

    hm3 = _tc_call(
        _tc_segmax, jax.ShapeDtypeStruct((NP, HP), _f32),
        h3, batch2)

    graph_logits, node_logits = _tc_call(
        _tc_final,
        (jax.ShapeDtypeStruct((G, 1), _f32),
         jax.ShapeDtypeStruct((1, NP), _f32)),
        h3, hm3, node_W, node_b.reshape(1, 1),
        gw1, gw2, graph_b.reshape(1, 1), batch2)

    return graph_logits.reshape(-1), node_logits.reshape(-1)[:N]


# speedup vs baseline: 27.6985x; 1.1695x over previous
import functools

import jax
import jax.numpy as jnp
from jax import lax
from jax.experimental import pallas as pl
from jax.experimental.pallas import tpu as pltpu
from jax.experimental.pallas import tpu_sc as plsc

N = 10000
NP = 10240
E = 320000
EP = 327680
D = 128
H = 96
HP = 128
G = 64

NC = 2
NS = 16
NW = NC * NS
EDGES_PER_TILE = EP // NW
CHUNK = 64
NCHUNK = EDGES_PER_TILE // CHUNK
NBUF = 4
IDXW = 128
ROWS_PER_TILE = NP // NS

_f32 = jnp.float32


def _sc_mesh():
    return plsc.VectorSubcoreMesh(core_axis_name="c", subcore_axis_name="s")


@functools.partial(
    pl.kernel,
    out_type=(
        jax.ShapeDtypeStruct((NP,), _f32),
        jax.ShapeDtypeStruct((NP,), _f32),
    ),
    mesh=_sc_mesh(),
    scratch_types=[
        pltpu.VMEM((NCHUNK, CHUNK), jnp.int32),
        pltpu.VMEM((CHUNK,), _f32),
        pltpu.VMEM_SHARED((NP,), _f32),
    ],
)
def _sc_degree(d_hbm, zeros_hbm, ones_hbm, deg_a, deg_b, d_idx, ones_v, deg_sh):
    c = lax.axis_index("c")
    s = lax.axis_index("s")
    wid = s * NC + c

    @pl.when(s == 0)
    def _():
        pltpu.sync_copy(zeros_hbm, deg_sh)
    pltpu.sync_copy(ones_hbm, ones_v)
    pltpu.sync_copy(d_hbm.at[pl.ds(wid * NCHUNK, NCHUNK)], d_idx)
    plsc.subcore_barrier()

    def body(j, carry):
        pltpu.sync_copy(ones_v, deg_sh.at[d_idx.at[j]], add=True)
        return carry

    lax.fori_loop(0, NCHUNK, body, 0)
    plsc.subcore_barrier()

    @pl.when((s == 0) & (c == 0))
    def _():
        pltpu.sync_copy(deg_sh, deg_a)

    @pl.when((s == 0) & (c == 1))
    def _():
        pltpu.sync_copy(deg_sh, deg_b)


@functools.partial(
    pl.kernel,
    out_type=(
        jax.ShapeDtypeStruct((NP, HP), _f32),
        jax.ShapeDtypeStruct((NP, HP), _f32),
    ),
    mesh=_sc_mesh(),
    scratch_types=[
        pltpu.VMEM((NCHUNK // 2, IDXW), jnp.int32),
        pltpu.VMEM((NBUF, CHUNK), jnp.int32),
        pltpu.VMEM((NBUF, CHUNK), jnp.int32),
        pltpu.VMEM((NBUF, CHUNK, HP), _f32),
        pltpu.VMEM_SHARED((NP, HP), _f32),
    ] + [pltpu.SemaphoreType.DMA] * (2 * NBUF),
)
def _sc_aggregate(p_hbm, zeros_hbm, sd_hbm, out_a, out_b,
                  sd_idx, s_buf, d_buf, rows, acc_sh, *sems):
    c = lax.axis_index("c")
    s = lax.axis_index("s")
    wid = s * NC + c
    row0 = s * ROWS_PER_TILE
    gsem = sems[:NBUF]
    ssem = sems[NBUF:]

    @pl.when(c == 0)
    def _():
        pltpu.sync_copy(p_hbm.at[pl.ds(row0, ROWS_PER_TILE)],
                        acc_sh.at[pl.ds(row0, ROWS_PER_TILE)])

    @pl.when(c == 1)
    def _():
        pltpu.sync_copy(zeros_hbm.at[pl.ds(row0, ROWS_PER_TILE)],
                        acc_sh.at[pl.ds(row0, ROWS_PER_TILE)])

    pltpu.sync_copy(sd_hbm.at[pl.ds(wid * (NCHUNK // 2), NCHUNK // 2)], sd_idx)
    plsc.subcore_barrier()

    def unpack(jrow, lane0, b):
        for q in range(CHUNK // 16):
            v = sd_idx[jrow, pl.ds(lane0 + q * 16, 16)]
            s_buf[b, pl.ds(q * 16, 16)] = lax.bitwise_and(v, 0x3FFF)
            d_buf[b, pl.ds(q * 16, 16)] = lax.shift_right_logical(v, 14)

    for b in range(NBUF - 1):
        unpack(b // 2, (b % 2) * CHUNK, b)
        pltpu.async_copy(p_hbm.at[s_buf.at[b]], rows.at[b], gsem[b])

    def group(i0, carry):
        for b in range(NBUF):
            j = NBUF * i0 + b
            pltpu.make_async_copy(
                p_hbm.at[s_buf.at[b]], rows.at[b], gsem[b]).wait()
            pltpu.async_copy(rows.at[b], acc_sh.at[d_buf.at[b]],
                             ssem[b], add=True)
            bn = (b + NBUF - 1) % NBUF

            def prep(i0=i0, j=j, b=b, bn=bn):
                def wait_prev():
                    pltpu.make_async_copy(
                        rows.at[bn], acc_sh.at[d_buf.at(bn)] if False
                        else acc_sh.at[d_buf.at[bn]], ssem[bn]).wait()

                if b == 0:
                    pl.when(i0 >= 1)(wait_prev)
                else:
                    wait_prev()
                unpack(2 * i0 + (b + 3) // 2, ((b + 3) % 2) * CHUNK, bn)
                pltpu.async_copy(
                    p_hbm.at[s_buf.at[bn]], rows.at[bn], gsem[bn])

            if b == 0:
                prep()
            else:
                pl.when(j + NBUF - 1 < NCHUNK)(prep)
        return carry

    lax.fori_loop(0, NCHUNK // NBUF, group, 0)
    for b in range(NBUF):
        pltpu.make_async_copy(
            rows.at[b], acc_sh.at[d_buf.at[b]], ssem[b]).wait()
    plsc.subcore_barrier()

    @pl.when(c == 0)
    def _():
        pltpu.sync_copy(acc_sh.at[pl.ds(row0, ROWS_PER_TILE)],
                        out_a.at[pl.ds(row0, ROWS_PER_TILE)])

    @pl.when(c == 1)
    def _():
        pltpu.sync_copy(acc_sh.at[pl.ds(row0, ROWS_PER_TILE)],
                        out_b.at[pl.ds(row0, ROWS_PER_TILE)])


def _tc_first(x_ref, w_ref, dega_ref, degb_ref, p_ref, dinv_ref):
    deg = dega_ref[...] + degb_ref[...] + 1.0
    dinv = lax.rsqrt(deg)
    dinv_ref[...] = dinv
    p_ref[...] = dinv * jnp.dot(x_ref[...], w_ref[...],
                                preferred_element_type=_f32)


def _tc_mid(aa_ref, ab_ref, dinv_ref, b_ref, w_ref, p_ref):
    dinv = dinv_ref[...]
    h = jax.nn.relu(dinv * (aa_ref[...] + ab_ref[...]) + b_ref[...])
    p_ref[...] = dinv * jnp.dot(h, w_ref[...], preferred_element_type=_f32)


def _tc_h3(aa_ref, ab_ref, dinv_ref, b_ref, h_ref):
    dinv = dinv_ref[...]
    h_ref[...] = jax.nn.relu(
        dinv * (aa_ref[...] + ab_ref[...]) + b_ref[...])


def _tc_segmax(h_ref, batch_ref, hm_ref):
    bm = batch_ref[...]
    hm = h_ref[...]
    k = 1
    while k < NP:
        hs = jnp.concatenate([jnp.zeros((k, HP), _f32), hm[:-k]], axis=0)
        bs = jnp.concatenate(
            [jnp.full((k, 1), -1, jnp.int32), bm[:-k]], axis=0)
        hm = jnp.where(bs == bm, jnp.maximum(hm, hs), hm)
        k *= 2
    hm_ref[...] = hm


def _tc_final(h_ref, hm_ref, nw_ref, nb_ref,
              gw1_ref, gw2_ref, gb_ref, batch_ref,
              graph_ref, node_ref):
    h = h_ref[...][:, :H]

    node_ref[...] = lax.dot_general(
        nw_ref[...], h, (((0,), (1,)), ((), ())),
        preferred_element_type=_f32) + nb_ref[...]

    bm = batch_ref[...]
    gids = lax.broadcasted_iota(jnp.int32, (1, G), 1)
    onehot = (bm == gids).astype(_f32)
    contract = (((0,), (0,)), ((), ()))
    sums = lax.dot_general(onehot, h, contract,
                           preferred_element_type=_f32)
    ones_col = jnp.ones((NP, 1), _f32)
    counts = lax.dot_general(onehot, ones_col, contract,
                             preferred_element_type=_f32)
    mean_p = sums / jnp.maximum(counts, 1.0)

    b_next = jnp.concatenate(
        [bm[1:], jnp.full((1, 1), -2, jnp.int32)], axis=0)
    is_end = (b_next != bm).astype(_f32)
    oh_end = onehot * is_end
    max_p = lax.dot_general(oh_end, hm_ref[...][:, :H], contract,
                            preferred_element_type=_f32)

    graph_ref[...] = (
        jnp.dot(mean_p, gw1_ref[...], preferred_element_type=_f32)
        + jnp.dot(max_p, gw2_ref[...], preferred_element_type=_f32)
        + gb_ref[...]
    )


def _tc_call(body, out_shapes, *args):
    return pl.pallas_call(body, out_shape=out_shapes)(*args)


def _pad_cols(w):
    return jnp.concatenate(
        [w, jnp.zeros((w.shape[0], HP - w.shape[1]), _f32)], axis=1)


def _pad_rows(w):
    return jnp.concatenate(
        [w, jnp.zeros((HP - w.shape[0], w.shape[1]), _f32)], axis=0)


@jax.jit
def kernel(x, edge_index, batch, W1, b1, W2, b2, W3, b3,
           node_W, node_b, graph_W, graph_b):
    npad = EP - E
    pad_s = (jnp.arange(npad, dtype=jnp.int32) * 131) % N
    pad_d = N + (jnp.arange(npad, dtype=jnp.int32) % (NP - N))
    s_full = jnp.concatenate([edge_index[0], pad_s])
    d_full = jnp.concatenate([edge_index[1], pad_d])
    sd_r = (s_full | (d_full << 14)).reshape(NW * NCHUNK // 2, IDXW)
    d_r = d_full.reshape(NW * NCHUNK, CHUNK)
    zeros_nh = jnp.zeros((NP, HP), _f32)
    zeros_n = jnp.zeros((NP,), _f32)
    ones_c = jnp.ones((CHUNK,), _f32)
    xp = jnp.concatenate([x, jnp.zeros((NP - N, D), _f32)], axis=0)
    batch2 = jnp.concatenate(
        [batch, jnp.full((NP - N,), -1, jnp.int32)]).reshape(NP, 1)
    W1p = _pad_cols(W1)
    W2p = _pad_rows(_pad_cols(W2))
    W3p = _pad_rows(_pad_cols(W3))
    b1p = _pad_cols(b1.reshape(1, H))
    b2p = _pad_cols(b2.reshape(1, H))
    b3p = _pad_cols(b3.reshape(1, H))
    gw1 = graph_W[:H]
    gw2 = graph_W[H:]

    deg_a, deg_b = _sc_degree(d_r, zeros_n, ones_c)

    p1, dinv = _tc_call(
        _tc_first,
        (jax.ShapeDtypeStruct((NP, HP), _f32),
         jax.ShapeDtypeStruct((NP, 1), _f32)),
        xp, W1p, deg_a.reshape(NP, 1), deg_b.reshape(NP, 1))

    a1, b1_ = _sc_aggregate(p1, zeros_nh, sd_r)
    p2 = _tc_call(
        _tc_mid, jax.ShapeDtypeStruct((NP, HP), _f32),
        a1, b1_, dinv, b1p, W2p)

    a2, b2_ = _sc_aggregate(p2, zeros_nh, sd_r)
    p3 = _tc_call(
        _tc_mid, jax.ShapeDtypeStruct((NP, HP), _f32),
        a2, b2_, dinv, b2p, W3p)

    a3, b3_ = _sc_aggregate(p3, zeros_nh, sd_r)
    h3 = _tc_call(
        _tc_h3, jax.ShapeDtypeStruct((NP, HP), _f32),
        a3, b3_, dinv, b3p)

    hm3 = _tc_call(
        _tc_segmax, jax.ShapeDtypeStruct((NP, HP), _f32),
        h3, batch2)

    graph_logits, node_logits = _tc_call(
        _tc_final,
        (jax.ShapeDtypeStruct((G, 1), _f32),
         jax.ShapeDtypeStruct((1, NP), _f32)),
        h3, hm3, node_W, node_b.reshape(1, 1),
        gw1, gw2, graph_b.reshape(1, 1), batch2)

    return graph_logits.reshape(-1), node_logits.reshape(-1)[:N]


# speedup vs baseline: 29.1666x; 1.0530x over previous
import functools

import jax
import jax.numpy as jnp
from jax import lax
from jax.experimental import pallas as pl
from jax.experimental.pallas import tpu as pltpu
from jax.experimental.pallas import tpu_sc as plsc

N = 10000
NP = 10240
E = 320000
EP = 327680
D = 128
H = 96
HP = 128
G = 64

NC = 2
NS = 16
NW = NC * NS
EDGES_PER_TILE = EP // NW
CHUNK = 32
NCHUNK = EDGES_PER_TILE // CHUNK
NBUF = 8
IDXW = 128
CPR = IDXW // CHUNK
ROWS_PER_TILE = NP // NS

_f32 = jnp.float32


def _sc_mesh():
    return plsc.VectorSubcoreMesh(core_axis_name="c", subcore_axis_name="s")


@functools.partial(
    pl.kernel,
    out_type=(
        jax.ShapeDtypeStruct((NP,), _f32),
        jax.ShapeDtypeStruct((NP,), _f32),
    ),
    mesh=_sc_mesh(),
    scratch_types=[
        pltpu.VMEM((NCHUNK, CHUNK), jnp.int32),
        pltpu.VMEM((CHUNK,), _f32),
        pltpu.VMEM_SHARED((NP,), _f32),
    ],
)
def _sc_degree(d_hbm, zeros_hbm, ones_hbm, deg_a, deg_b, d_idx, ones_v, deg_sh):
    c = lax.axis_index("c")
    s = lax.axis_index("s")
    wid = s * NC + c

    @pl.when(s == 0)
    def _():
        pltpu.sync_copy(zeros_hbm, deg_sh)
    pltpu.sync_copy(ones_hbm, ones_v)
    pltpu.sync_copy(d_hbm.at[pl.ds(wid * NCHUNK, NCHUNK)], d_idx)
    plsc.subcore_barrier()

    def body(j, carry):
        pltpu.sync_copy(ones_v, deg_sh.at[d_idx.at[j]], add=True)
        return carry

    lax.fori_loop(0, NCHUNK, body, 0)
    plsc.subcore_barrier()

    @pl.when((s == 0) & (c == 0))
    def _():
        pltpu.sync_copy(deg_sh, deg_a)

    @pl.when((s == 0) & (c == 1))
    def _():
        pltpu.sync_copy(deg_sh, deg_b)


@functools.partial(
    pl.kernel,
    out_type=(
        jax.ShapeDtypeStruct((NP, HP), _f32),
        jax.ShapeDtypeStruct((NP, HP), _f32),
    ),
    mesh=_sc_mesh(),
    scratch_types=[
        pltpu.VMEM((NCHUNK // CPR, IDXW), jnp.int32),
        pltpu.VMEM((NBUF, CHUNK), jnp.int32),
        pltpu.VMEM((NBUF, CHUNK), jnp.int32),
        pltpu.VMEM((NBUF, CHUNK, HP), _f32),
        pltpu.VMEM_SHARED((NP, HP), _f32),
    ] + [pltpu.SemaphoreType.DMA] * (2 * NBUF),
)
def _sc_aggregate(p_hbm, zeros_hbm, sd_hbm, out_a, out_b,
                  sd_idx, s_buf, d_buf, rows, acc_sh, *sems):
    c = lax.axis_index("c")
    s = lax.axis_index("s")
    wid = s * NC + c
    row0 = s * ROWS_PER_TILE
    gsem = sems[:NBUF]
    ssem = sems[NBUF:]

    @pl.when(c == 0)
    def _():
        pltpu.sync_copy(p_hbm.at[pl.ds(row0, ROWS_PER_TILE)],
                        acc_sh.at[pl.ds(row0, ROWS_PER_TILE)])

    @pl.when(c == 1)
    def _():
        pltpu.sync_copy(zeros_hbm.at[pl.ds(row0, ROWS_PER_TILE)],
                        acc_sh.at[pl.ds(row0, ROWS_PER_TILE)])

    pltpu.sync_copy(sd_hbm.at[pl.ds(wid * (NCHUNK // CPR), NCHUNK // CPR)], sd_idx)
    plsc.subcore_barrier()

    def unpack(jrow, lane0, b):
        for q in range(CHUNK // 16):
            v = sd_idx[jrow, pl.ds(lane0 + q * 16, 16)]
            s_buf[b, pl.ds(q * 16, 16)] = lax.bitwise_and(v, 0x3FFF)
            d_buf[b, pl.ds(q * 16, 16)] = lax.shift_right_logical(v, 14)

    for b in range(NBUF - 1):
        unpack(b // CPR, (b % CPR) * CHUNK, b)
        pltpu.async_copy(p_hbm.at[s_buf.at[b]], rows.at[b], gsem[b])

    def group(i0, carry):
        for b in range(NBUF):
            j = NBUF * i0 + b
            pltpu.make_async_copy(
                p_hbm.at[s_buf.at[b]], rows.at[b], gsem[b]).wait()
            pltpu.async_copy(rows.at[b], acc_sh.at[d_buf.at[b]],
                             ssem[b], add=True)
            bn = (b + NBUF - 1) % NBUF

            def prep(i0=i0, j=j, b=b, bn=bn):
                def wait_prev():
                    pltpu.make_async_copy(
                        rows.at[bn], acc_sh.at[d_buf.at(bn)] if False
                        else acc_sh.at[d_buf.at[bn]], ssem[bn]).wait()

                if b == 0:
                    pl.when(i0 >= 1)(wait_prev)
                else:
                    wait_prev()
                unpack((NBUF // CPR) * i0 + (b + NBUF - 1) // CPR,
                       ((b + NBUF - 1) % CPR) * CHUNK, bn)
                pltpu.async_copy(
                    p_hbm.at[s_buf.at[bn]], rows.at[bn], gsem[bn])

            if b == 0:
                prep()
            else:
                pl.when(j + NBUF - 1 < NCHUNK)(prep)
        return carry

    lax.fori_loop(0, NCHUNK // NBUF, group, 0)
    for b in range(NBUF):
        pltpu.make_async_copy(
            rows.at[b], acc_sh.at[d_buf.at[b]], ssem[b]).wait()
    plsc.subcore_barrier()

    @pl.when(c == 0)
    def _():
        pltpu.sync_copy(acc_sh.at[pl.ds(row0, ROWS_PER_TILE)],
                        out_a.at[pl.ds(row0, ROWS_PER_TILE)])

    @pl.when(c == 1)
    def _():
        pltpu.sync_copy(acc_sh.at[pl.ds(row0, ROWS_PER_TILE)],
                        out_b.at[pl.ds(row0, ROWS_PER_TILE)])


def _tc_first(x_ref, w_ref, dega_ref, degb_ref, p_ref, dinv_ref):
    deg = dega_ref[...] + degb_ref[...] + 1.0
    dinv = lax.rsqrt(deg)
    dinv_ref[...] = dinv
    p_ref[...] = dinv * jnp.dot(x_ref[...], w_ref[...],
                                preferred_element_type=_f32)


def _tc_mid(aa_ref, ab_ref, dinv_ref, b_ref, w_ref, p_ref):
    dinv = dinv_ref[...]
    h = jax.nn.relu(dinv * (aa_ref[...] + ab_ref[...]) + b_ref[...])
    p_ref[...] = dinv * jnp.dot(h, w_ref[...], preferred_element_type=_f32)


def _tc_h3(aa_ref, ab_ref, dinv_ref, b_ref, h_ref):
    dinv = dinv_ref[...]
    h_ref[...] = jax.nn.relu(
        dinv * (aa_ref[...] + ab_ref[...]) + b_ref[...])


def _tc_segmax(h_ref, batch_ref, hm_ref):
    bm = batch_ref[...]
    hm = h_ref[...]
    k = 1
    while k < NP:
        hs = jnp.concatenate([jnp.zeros((k, HP), _f32), hm[:-k]], axis=0)
        bs = jnp.concatenate(
            [jnp.full((k, 1), -1, jnp.int32), bm[:-k]], axis=0)
        hm = jnp.where(bs == bm, jnp.maximum(hm, hs), hm)
        k *= 2
    hm_ref[...] = hm


def _tc_final(h_ref, hm_ref, nw_ref, nb_ref,
              gw1_ref, gw2_ref, gb_ref, batch_ref,
              graph_ref, node_ref):
    h = h_ref[...][:, :H]

    node_ref[...] = lax.dot_general(
        nw_ref[...], h, (((0,), (1,)), ((), ())),
        preferred_element_type=_f32) + nb_ref[...]

    bm = batch_ref[...]
    gids = lax.broadcasted_iota(jnp.int32, (1, G), 1)
    onehot = (bm == gids).astype(_f32)
    contract = (((0,), (0,)), ((), ()))
    sums = lax.dot_general(onehot, h, contract,
                           preferred_element_type=_f32)
    ones_col = jnp.ones((NP, 1), _f32)
    counts = lax.dot_general(onehot, ones_col, contract,
                             preferred_element_type=_f32)
    mean_p = sums / jnp.maximum(counts, 1.0)

    b_next = jnp.concatenate(
        [bm[1:], jnp.full((1, 1), -2, jnp.int32)], axis=0)
    is_end = (b_next != bm).astype(_f32)
    oh_end = onehot * is_end
    max_p = lax.dot_general(oh_end, hm_ref[...][:, :H], contract,
                            preferred_element_type=_f32)

    graph_ref[...] = (
        jnp.dot(mean_p, gw1_ref[...], preferred_element_type=_f32)
        + jnp.dot(max_p, gw2_ref[...], preferred_element_type=_f32)
        + gb_ref[...]
    )


def _tc_call(body, out_shapes, *args):
    return pl.pallas_call(body, out_shape=out_shapes)(*args)


def _pad_cols(w):
    return jnp.concatenate(
        [w, jnp.zeros((w.shape[0], HP - w.shape[1]), _f32)], axis=1)


def _pad_rows(w):
    return jnp.concatenate(
        [w, jnp.zeros((HP - w.shape[0], w.shape[1]), _f32)], axis=0)


@jax.jit
def kernel(x, edge_index, batch, W1, b1, W2, b2, W3, b3,
           node_W, node_b, graph_W, graph_b):
    npad = EP - E
    pad_s = (jnp.arange(npad, dtype=jnp.int32) * 131) % N
    pad_d = N + (jnp.arange(npad, dtype=jnp.int32) % (NP - N))
    s_full = jnp.concatenate([edge_index[0], pad_s])
    d_full = jnp.concatenate([edge_index[1], pad_d])
    sd_r = (s_full | (d_full << 14)).reshape(NW * (NCHUNK // CPR), IDXW)
    d_r = d_full.reshape(NW * NCHUNK, CHUNK)
    zeros_nh = jnp.zeros((NP, HP), _f32)
    zeros_n = jnp.zeros((NP,), _f32)
    ones_c = jnp.ones((CHUNK,), _f32)
    xp = jnp.concatenate([x, jnp.zeros((NP - N, D), _f32)], axis=0)
    batch2 = jnp.concatenate(
        [batch, jnp.full((NP - N,), -1, jnp.int32)]).reshape(NP, 1)
    W1p = _pad_cols(W1)
    W2p = _pad_rows(_pad_cols(W2))
    W3p = _pad_rows(_pad_cols(W3))
    b1p = _pad_cols(b1.reshape(1, H))
    b2p = _pad_cols(b2.reshape(1, H))
    b3p = _pad_cols(b3.reshape(1, H))
    gw1 = graph_W[:H]
    gw2 = graph_W[H:]

    deg_a, deg_b = _sc_degree(d_r, zeros_n, ones_c)

    p1, dinv = _tc_call(
        _tc_first,
        (jax.ShapeDtypeStruct((NP, HP), _f32),
         jax.ShapeDtypeStruct((NP, 1), _f32)),
        xp, W1p, deg_a.reshape(NP, 1), deg_b.reshape(NP, 1))

    a1, b1_ = _sc_aggregate(p1, zeros_nh, sd_r)
    p2 = _tc_call(
        _tc_mid, jax.ShapeDtypeStruct((NP, HP), _f32),
        a1, b1_, dinv, b1p, W2p)

    a2, b2_ = _sc_aggregate(p2, zeros_nh, sd_r)
    p3 = _tc_call(
        _tc_mid, jax.ShapeDtypeStruct((NP, HP), _f32),
        a2, b2_, dinv, b2p, W3p)

    a3, b3_ = _sc_aggregate(p3, zeros_nh, sd_r)
    h3 = _tc_call(
        _tc_h3, jax.ShapeDtypeStruct((NP, HP), _f32),
        a3, b3_, dinv, b3p)

    hm3 = _tc_call(
        _tc_segmax, jax.ShapeDtypeStruct((NP, HP), _f32),
        h3, batch2)

    graph_logits, node_logits = _tc_call(
        _tc_final,
        (jax.ShapeDtypeStruct((G, 1), _f32),
         jax.ShapeDtypeStruct((1, NP), _f32)),
        h3, hm3, node_W, node_b.reshape(1, 1),
        gw1, gw2, graph_b.reshape(1, 1), batch2)

    return graph_logits.reshape(-1), node_logits.reshape(-1)[:N]
